# baseline (speedup 1.0000x reference)
import jax
import jax.numpy as jnp
import numpy as np
from jax import lax
from jax.experimental import pallas as pl
from jax.experimental.pallas import tpu as pltpu
from jax.experimental.pallas import tpu_sc as plsc

B = 16384
NCAT = 26
DNUM = 13
DT = 32
NTOK = 1 + DNUM + NCAT
NC = 2
NS = 16
NW = NC * NS
BPW = B // NW
NB = 8
NCHUNK = BPW // NB
F = NB * NCAT
G = 104
NG = F // G
PERIOD = 208
SR = 128
TBL4 = 2600000 * DT // SR
ORPC = NB * NTOK * DT // SR
NOUT = B * NTOK * DT // SR


def _body(xnum_hbm, xcat_hbm, w_hbm, b_hbm, table_hbm, offs_hbm, out_hbm,
          xcat_v, xnum_v, w_v, b_v, offs_v,
          idx_v, q_v, temp_v, obuf_v, sem, osem):
    cid = lax.axis_index("c")
    sid = lax.axis_index("s")
    wid = sid * NC + cid
    bb0 = wid * BPW
    fb0 = bb0 * NCAT

    pltpu.sync_copy(xcat_hbm.at[pl.ds(fb0, BPW * NCAT)], xcat_v)
    pltpu.sync_copy(xnum_hbm.at[:, pl.ds(bb0, BPW)],
                    xnum_v.at[:, pl.ds(0, BPW)])
    pltpu.sync_copy(w_hbm, w_v)
    pltpu.sync_copy(b_hbm, b_v)
    pltpu.sync_copy(offs_hbm, offs_v)

    def chunk2(t2, carry):
      xvall = [xnum_v[j, pl.ds(t2 * 16, 16)] for j in range(DNUM)]
      for s in range(2):
        t = t2 * 2 + s
        f0 = t * F
        buf = s

        @pl.when(t >= 2)
        def _():
            pltpu.make_async_copy(
                obuf_v.at[buf],
                out_hbm.at[pl.ds(pl.multiple_of((bb0 + (t - 2) * NB) * NTOK // 4, 8), ORPC)],
                osem).wait()

        for o in range(0, F, 16):
            gidx = (xcat_v[pl.ds(f0 + o, 16)]
                    + offs_v[pl.ds(o % PERIOD, 16)])
            idx_v[pl.ds(o, 16)] = lax.shift_right_logical(gidx, 2)
            q_v[pl.ds(o, 16)] = lax.bitwise_and(gidx, 3)

        handles = [
            pltpu.async_copy(table_hbm.at[idx_v.at[pl.ds(g * G, G)]],
                             temp_v.at[pl.ds(g * G, G)], sem)
            for g in range(NG)
        ]

        for bb in range(NB):
            p0 = bb * NTOK
            for h2 in range(2):
                obuf_v[buf, p0 // 4, pl.ds((p0 % 4) * DT + h2 * 16, 16)] = (
                    w_v[0, pl.ds(h2 * 16, 16)])
            for j in range(1, DNUM + 1):
                xs = xvall[j - 1][s * NB + bb]
                for h2 in range(2):
                    sl = pl.ds(h2 * 16, 16)
                    p = p0 + j
                    obuf_v[buf, p // 4, pl.ds((p % 4) * DT + h2 * 16, 16)] = (
                        w_v[j, sl] * xs + b_v[j - 1, sl])

        for h in handles:
            h.wait()

        for r0 in range(0, F, 16):
            qv = q_v[pl.ds(r0, 16)]
            for l in range(16):
                r = r0 + l
                bb, c = r // NCAT, r % NCAT
                qs = jnp.full((16,), qv[l], jnp.int32)
                masks = []
                for q in range(4):
                    dq = qs - q
                    masks.append(
                        (1 - jnp.minimum(dq * dq, 1)).astype(jnp.float32))
                p = bb * NTOK + 1 + DNUM + c
                for h2 in range(2):
                    quarters = [temp_v[r, pl.ds(q * DT + h2 * 16, 16)]
                                for q in range(4)]
                    sel = (quarters[0] * masks[0] + quarters[1] * masks[1]
                           + quarters[2] * masks[2] + quarters[3] * masks[3])
                    obuf_v[buf, p // 4, pl.ds((p % 4) * DT + h2 * 16, 16)] = (
                        sel + b_v[13 + c, pl.ds(h2 * 16, 16)])

        pltpu.async_copy(
            obuf_v.at[buf],
            out_hbm.at[pl.ds(pl.multiple_of((bb0 + t * NB) * NTOK // 4, 8), ORPC)], osem)
      return carry

    lax.fori_loop(0, NCHUNK // 2, chunk2, 0)
    for u in range(2):
        t = NCHUNK - 2 + u
        pltpu.make_async_copy(
            obuf_v.at[t % 2],
            out_hbm.at[pl.ds(pl.multiple_of((bb0 + t * NB) * NTOK // 4, 8), ORPC)], osem).wait()


def kernel(x_num, x_cat, weight, bias, emb_table, category_offsets):
    xcat_flat = x_cat.reshape(-1)
    offs_pat = jnp.tile(category_offsets, PERIOD // NCAT)
    table4 = emb_table.reshape(TBL4, SR)

    kfn = pl.kernel(
        _body,
        out_type=jax.ShapeDtypeStruct((NOUT, SR), jnp.float32),
        mesh=plsc.VectorSubcoreMesh(core_axis_name="c", subcore_axis_name="s"),
        scratch_types=[
            pltpu.VMEM((BPW * NCAT,), jnp.int32),
            pltpu.VMEM((DNUM, BPW + 16), jnp.float32),
            pltpu.VMEM((DNUM + 1, DT), jnp.float32),
            pltpu.VMEM((DNUM + NCAT, DT), jnp.float32),
            pltpu.VMEM((PERIOD,), jnp.int32),
            pltpu.VMEM((F,), jnp.int32),
            pltpu.VMEM((F,), jnp.int32),
            pltpu.VMEM((F, SR), jnp.float32),
            pltpu.VMEM((2, ORPC, SR), jnp.float32),
            pltpu.SemaphoreType.DMA,
            pltpu.SemaphoreType.DMA,
        ],
    )
    out = kfn(x_num.T, xcat_flat, weight, bias, table4, offs_pat)
    return out.reshape(B, NTOK, DT)


# speedup vs baseline: 1.2574x; 1.2574x over previous
import jax
import jax.numpy as jnp
import numpy as np
from jax import lax
from jax.experimental import pallas as pl
from jax.experimental.pallas import tpu as pltpu
from jax.experimental.pallas import tpu_sc as plsc

B = 16384
NCAT = 26
DNUM = 13
DT = 32
NTOK = 1 + DNUM + NCAT
NC = 2
NS = 16
NW = NC * NS
BPW = B // NW
NB = 16
NCHUNK = BPW // NB
F = NB * NCAT
G = 104
NG = F // G
PERIOD = 208
SR = 128
ORPC = NB * NTOK
NOUT = B * NTOK


def _body(xnum_hbm, xcat_hbm, w_hbm, b_hbm, table_hbm, offs_hbm, out_hbm,
          xcat_v, xnum_v, w_v, b_v, offs_v,
          idx_v, temp_v, obuf_v, sem, osem):
    cid = lax.axis_index("c")
    sid = lax.axis_index("s")
    wid = sid * NC + cid
    bb0 = wid * BPW
    fb0 = bb0 * NCAT

    pltpu.sync_copy(xcat_hbm.at[pl.ds(fb0, BPW * NCAT)], xcat_v)
    pltpu.sync_copy(xnum_hbm.at[:, pl.ds(bb0, BPW)], xnum_v)
    pltpu.sync_copy(w_hbm, w_v)
    pltpu.sync_copy(b_hbm, b_v)
    pltpu.sync_copy(offs_hbm, offs_v)

    def compute_idx(t, buf):
        f0 = t * F
        for o in range(0, F, 16):
            idx_v[buf, pl.ds(o, 16)] = (
                xcat_v[pl.ds(f0 + o, 16)] + offs_v[pl.ds(o % PERIOD, 16)])

    def fire_gathers(buf):
        for g in range(NG):
            pltpu.async_copy(table_hbm.at[idx_v.at[buf, pl.ds(g * G, G)]],
                             temp_v.at[buf, pl.ds(g * G, G)], sem)

    def drain_gathers(buf):
        for g in range(NG):
            pltpu.make_async_copy(
                table_hbm.at[idx_v.at[buf, pl.ds(g * G, G)]],
                temp_v.at[buf, pl.ds(g * G, G)], sem).wait()

    compute_idx(0, 0)
    fire_gathers(0)

    def chunk(t, carry):
        buf = lax.rem(t, 2)
        nbuf = lax.rem(t + 1, 2)

        @pl.when(t >= 2)
        def _():
            pltpu.make_async_copy(
                obuf_v.at[buf],
                out_hbm.at[pl.ds((bb0 + (t - 2) * NB) * NTOK, ORPC),
                           pl.ds(0, DT)],
                osem).wait()

        xvs = [xnum_v[j, pl.ds(t * NB, 16)] for j in range(DNUM)]
        for bb in range(NB):
            p0 = bb * NTOK
            for h2 in range(2):
                sl = pl.ds(h2 * 16, 16)
                obuf_v[buf, p0, sl] = w_v[0, sl]
            for j in range(1, DNUM + 1):
                xs = xvs[j - 1][bb]
                for h2 in range(2):
                    sl = pl.ds(h2 * 16, 16)
                    obuf_v[buf, p0 + j, sl] = w_v[j, sl] * xs + b_v[j - 1, sl]

        @pl.when(t + 1 < NCHUNK)
        def _():
            compute_idx(t + 1, nbuf)
            fire_gathers(nbuf)

        drain_gathers(buf)

        def biasb(bb, c2):
            r = bb * NCAT
            p0 = bb * NTOK + 1 + DNUM
            for c in range(NCAT):
                for h2 in range(2):
                    sl = pl.ds(h2 * 16, 16)
                    obuf_v[buf, p0 + c, sl] = (
                        temp_v[buf, r + c, sl] + b_v[13 + c, sl])
            return c2

        lax.fori_loop(0, NB, biasb, 0)

        pltpu.async_copy(
            obuf_v.at[buf],
            out_hbm.at[pl.ds((bb0 + t * NB) * NTOK, ORPC), pl.ds(0, DT)],
            osem)
        return carry

    lax.fori_loop(0, NCHUNK, chunk, 0)
    for u in range(2):
        t = NCHUNK - 2 + u
        pltpu.make_async_copy(
            obuf_v.at[t % 2],
            out_hbm.at[pl.ds((bb0 + t * NB) * NTOK, ORPC), pl.ds(0, DT)],
            osem).wait()


def kernel(x_num, x_cat, weight, bias, emb_table, category_offsets):
    xcat_flat = x_cat.reshape(-1)
    offs_pat = jnp.tile(category_offsets, PERIOD // NCAT)

    kfn = pl.kernel(
        _body,
        out_type=jax.ShapeDtypeStruct((NOUT, SR), jnp.float32),
        mesh=plsc.VectorSubcoreMesh(core_axis_name="c", subcore_axis_name="s"),
        compiler_params=pltpu.CompilerParams(use_tc_tiling_on_sc=False),
        scratch_types=[
            pltpu.VMEM((BPW * NCAT,), jnp.int32),
            pltpu.VMEM((DNUM, BPW), jnp.float32),
            pltpu.VMEM((DNUM + 1, DT), jnp.float32),
            pltpu.VMEM((DNUM + NCAT, DT), jnp.float32),
            pltpu.VMEM((PERIOD,), jnp.int32),
            pltpu.VMEM((2, F), jnp.int32),
            pltpu.VMEM((2, F, DT), jnp.float32),
            pltpu.VMEM((2, ORPC, DT), jnp.float32),
            pltpu.SemaphoreType.DMA,
            pltpu.SemaphoreType.DMA,
        ],
    )
    out = kfn(x_num.T, xcat_flat, weight, bias, emb_table, offs_pat)
    return out[:, :DT].reshape(B, NTOK, DT)
